# Initial kernel scaffold; baseline (speedup 1.0000x reference)
import jax
import jax.numpy as jnp
from jax.experimental import pallas as pl


def kernel(t, h, edge_index_pos, edge_index_neg, gamma, beta, Wp_self, Wp_nbr, Wn_self, Wn_nbr, W_psi):
    raise NotImplementedError("write your pallas kernel here")


# speedup vs baseline: 4.3962x; 4.3962x over previous
import functools

import jax
import jax.numpy as jnp
from jax import lax
from jax.experimental import pallas as pl
from jax.experimental.pallas import tpu as pltpu
from jax.experimental.pallas import tpu_sc as plsc

N = 100000
D = 32
E = 1600000
DAMPING = 0.1
EPS = 1e-5

BE = 128
TB = E // BE
NSUB = 16
QB, RB = divmod(TB, NSUB)
HALF_TB = TB // 2
ZROWS = 6272
N_PAD = NSUB * ZROWS
DROWS_A = 6248
DROWS_B = N - 15 * DROWS_A


_LN_BLK = 2000


def _ln_body(h_ref, g_ref, b_ref, out_ref):
    x = h_ref[...]
    mean = jnp.mean(x, axis=1, keepdims=True)
    xc = x - mean
    var = jnp.mean(xc * xc, axis=1, keepdims=True)
    y = xc * jax.lax.rsqrt(var + EPS) * g_ref[...] + b_ref[...]
    out_ref[0] = y[:, :16]
    out_ref[1] = y[:, 16:]


def _layernorm_split(h, gamma, beta):
    grid = N // _LN_BLK
    out = pl.pallas_call(
        _ln_body,
        grid=(grid,),
        in_specs=[
            pl.BlockSpec((_LN_BLK, D), lambda i: (i, 0)),
            pl.BlockSpec((1, D), lambda i: (0, 0)),
            pl.BlockSpec((1, D), lambda i: (0, 0)),
        ],
        out_specs=pl.BlockSpec((2, _LN_BLK, 16), lambda i: (0, i, 0)),
        out_shape=jax.ShapeDtypeStruct((2, N, 16), jnp.float32),
    )(h, gamma.reshape(1, D), beta.reshape(1, D))
    return out.reshape(2 * N, 16)


def _conv_body(hn2, src, dst, z16, z1, agg_out, deg_out,
               table_sh, deg_sh, sbuf, dbuf, rows, ones_v):
    c = lax.axis_index("c")
    s = lax.axis_index("s")

    zbase = s * ZROWS
    pltpu.sync_copy(z16, table_sh.at[pl.ds(zbase, ZROWS)])
    pltpu.sync_copy(z1, deg_sh.at[pl.ds(zbase, ZROWS)])
    for j in range(BE // 16):
        ones_v[pl.ds(j * 16, 16)] = jnp.full((16,), 1.0, jnp.float32)
    plsc.subcore_barrier()

    start_b = s * QB + jnp.minimum(s, RB)
    count = QB + jnp.where(s < RB, 1, 0)
    coff = c * N
    deg_lo = c == 0

    def body(gb, _):
        eoff = gb * BE
        pltpu.sync_copy(src.at[pl.ds(eoff, BE)], sbuf)
        pltpu.sync_copy(dst.at[pl.ds(eoff, BE)], dbuf)
        for j in range(BE // 16):
            sl = pl.ds(j * 16, 16)
            sbuf[sl] = sbuf[sl] + coff
        pltpu.sync_copy(hn2.at[sbuf], rows)
        pltpu.sync_copy(rows, table_sh.at[dbuf], add=True)

        @pl.when((gb < HALF_TB) == deg_lo)
        def _():
            pltpu.sync_copy(ones_v, deg_sh.at[dbuf], add=True)

        return _

    lax.fori_loop(start_b, start_b + count, body, None)
    plsc.subcore_barrier()

    @pl.when(s < 15)
    def _():
        st = s * DROWS_A
        pltpu.sync_copy(table_sh.at[pl.ds(st, DROWS_A)],
                        agg_out.at[pl.ds(coff + st, DROWS_A)])
        pltpu.sync_copy(deg_sh.at[pl.ds(st, DROWS_A)],
                        deg_out.at[pl.ds(coff + st, DROWS_A)])

    @pl.when(s == 15)
    def _():
        st = 15 * DROWS_A
        pltpu.sync_copy(table_sh.at[pl.ds(st, DROWS_B)],
                        agg_out.at[pl.ds(coff + st, DROWS_B)])
        pltpu.sync_copy(deg_sh.at[pl.ds(st, DROWS_B)],
                        deg_out.at[pl.ds(coff + st, DROWS_B)])


_conv_sc = pl.kernel(
    _conv_body,
    out_type=(
        jax.ShapeDtypeStruct((2 * N, 16), jnp.float32),
        jax.ShapeDtypeStruct((2 * N,), jnp.float32),
    ),
    mesh=plsc.VectorSubcoreMesh(core_axis_name="c", subcore_axis_name="s"),
    compiler_params=pltpu.CompilerParams(use_tc_tiling_on_sc=False),
    scratch_types=[
        pltpu.VMEM_SHARED((N_PAD, 16), jnp.float32),
        pltpu.VMEM_SHARED((N_PAD,), jnp.float32),
        pltpu.VMEM((BE,), jnp.int32),
        pltpu.VMEM((BE,), jnp.int32),
        pltpu.VMEM((BE, 16), jnp.float32),
        pltpu.VMEM((BE,), jnp.float32),
    ],
)


_F_BLK = 2000


def _fuse_body(hs_ref, ap_ref, an_ref, dp_ref, dn_ref,
               wps_ref, wpn_ref, wns_ref, wnn_ref, wpsi_ref, out_ref):
    f32 = jnp.float32
    hn = jnp.concatenate([hs_ref[0], hs_ref[1]], axis=1)
    degp = jnp.maximum(dp_ref[0] + dp_ref[1], 1.0)
    degn = jnp.maximum(dn_ref[0] + dn_ref[1], 1.0)
    aggp = jnp.concatenate([ap_ref[0], ap_ref[1]], axis=1) / degp
    aggn = jnp.concatenate([an_ref[0], an_ref[1]], axis=1) / degn

    dot = functools.partial(lax.dot_general, precision=lax.Precision.HIGHEST,
                            preferred_element_type=f32)
    mm = lambda a, b: dot(a, b, (((1,), (0,)), ((), ())))
    mmt = lambda a, b: dot(a, b, (((1,), (1,)), ((), ())))

    wpsi = wpsi_ref[...]
    hp = mm(hn, wps_ref[...]) + mm(aggp, wpn_ref[...])
    hm = mm(hn, wns_ref[...]) + mm(aggn, wnn_ref[...])
    z = mmt(hp, wpsi[:, :D]) + mmt(hm, wpsi[:, D:])
    out_ref[...] = jnp.clip(jnp.tanh(z) - DAMPING * hn, -50.0, 50.0)


def _fuse(hn2, aggp, degp, aggn, degn, Wp_self, Wp_nbr, Wn_self, Wn_nbr, W_psi):
    grid = N // _F_BLK
    split3 = pl.BlockSpec((2, _F_BLK, 16), lambda i: (0, i, 0))
    split2 = pl.BlockSpec((2, _F_BLK, 1), lambda i: (0, i, 0))
    wspec = pl.BlockSpec((D, D), lambda i: (0, 0))
    return pl.pallas_call(
        _fuse_body,
        grid=(grid,),
        in_specs=[split3, split3, split3, split2, split2,
                  wspec, wspec, wspec, wspec,
                  pl.BlockSpec((D, 2 * D), lambda i: (0, 0))],
        out_specs=pl.BlockSpec((_F_BLK, D), lambda i: (i, 0)),
        out_shape=jax.ShapeDtypeStruct((N, D), jnp.float32),
    )(hn2.reshape(2, N, 16), aggp.reshape(2, N, 16), aggn.reshape(2, N, 16),
      degp.reshape(2, N, 1), degn.reshape(2, N, 1),
      Wp_self, Wp_nbr, Wn_self, Wn_nbr, W_psi)


def kernel(t, h, edge_index_pos, edge_index_neg, gamma, beta,
           Wp_self, Wp_nbr, Wn_self, Wn_nbr, W_psi):
    hn2 = _layernorm_split(h, gamma, beta)
    z16 = jnp.zeros((ZROWS, 16), jnp.float32)
    z1 = jnp.zeros((ZROWS,), jnp.float32)
    aggp, degp = _conv_sc(hn2, edge_index_pos[0], edge_index_pos[1], z16, z1)
    aggn, degn = _conv_sc(hn2, edge_index_neg[0], edge_index_neg[1], z16, z1)
    return _fuse(hn2, aggp, degp, aggn, degn,
                 Wp_self, Wp_nbr, Wn_self, Wn_nbr, W_psi)


# speedup vs baseline: 9.4051x; 2.1394x over previous
import functools

import jax
import jax.numpy as jnp
from jax import lax
from jax.experimental import pallas as pl
from jax.experimental.pallas import tpu as pltpu
from jax.experimental.pallas import tpu_sc as plsc

N = 100000
D = 32
E = 1600000
DAMPING = 0.1
EPS = 1e-5

BE = 80
CHKB = 25
CHK_E = BE * CHKB
NSUB = 16
EPT = E // NSUB
NCHK = EPT // CHK_E
DPT = E // 2 // NSUB
NDCHK = DPT // CHK_E
NBUF = 4
ZROWS = 6272
N_PAD = NSUB * ZROWS
DROWS_A = 6248
DROWS_B = N - 15 * DROWS_A


_LN_BLK = 2000


def _ln_body(h_ref, g_ref, b_ref, out_ref):
    x = h_ref[...]
    mean = jnp.mean(x, axis=1, keepdims=True)
    xc = x - mean
    var = jnp.mean(xc * xc, axis=1, keepdims=True)
    y = xc * jax.lax.rsqrt(var + EPS) * g_ref[...] + b_ref[...]
    out_ref[0] = y[:, :16]
    out_ref[1] = y[:, 16:]


def _layernorm_split(h, gamma, beta):
    grid = N // _LN_BLK
    out = pl.pallas_call(
        _ln_body,
        grid=(grid,),
        in_specs=[
            pl.BlockSpec((_LN_BLK, D), lambda i: (i, 0)),
            pl.BlockSpec((1, D), lambda i: (0, 0)),
            pl.BlockSpec((1, D), lambda i: (0, 0)),
        ],
        out_specs=pl.BlockSpec((2, _LN_BLK, 16), lambda i: (0, i, 0)),
        out_shape=jax.ShapeDtypeStruct((2, N, 16), jnp.float32),
    )(h, gamma.reshape(1, D), beta.reshape(1, D))
    return out.reshape(2 * N, 16)


def _conv_body(hn2, eidx, z16, z1, agg_out, deg_out,
               table_sh, deg_sh, sbuf, dbuf, rows, ones_v,
               g0, g1, g2, g3, s0, s1, s2, s3):
    gsem = [g0, g1, g2, g3]
    ssem = [s0, s1, s2, s3]
    c = lax.axis_index("c")
    s = lax.axis_index("s")

    zbase = s * ZROWS
    pltpu.sync_copy(z16, table_sh.at[pl.ds(zbase, ZROWS)])
    pltpu.sync_copy(z1, deg_sh.at[pl.ds(zbase, ZROWS)])
    for j in range(BE // 16):
        ones_v[pl.ds(j * 16, 16)] = jnp.full((16,), 1.0, jnp.float32)
    plsc.subcore_barrier()

    coff = c * N
    tile_row0 = s * (EPT // BE)

    def main_chunk(k, carry):
        row0 = tile_row0 + k * CHKB
        pltpu.sync_copy(eidx.at[0, pl.ds(row0, CHKB)], sbuf)
        pltpu.sync_copy(eidx.at[1, pl.ds(row0, CHKB)], dbuf)
        for r in range(CHKB):
            for q in range(BE // 16):
                sl = pl.ds(q * 16, 16)
                sbuf[r, sl] = sbuf[r, sl] + coff
        g_desc = [None] * NBUF
        s_desc = [None] * NBUF

        def gather_start(j):
            p = j % NBUF
            g_desc[p] = pltpu.async_copy(
                hn2.at[sbuf.at[j]], rows.at[p], gsem[p])

        gather_start(0)
        gather_start(1)
        for j in range(CHKB):
            p = j % NBUF
            pn = (j + 2) % NBUF
            if j >= 2:
                s_desc[pn].wait()
            if j + 2 < CHKB:
                gather_start(j + 2)
            g_desc[p].wait()
            s_desc[p] = pltpu.async_copy(
                rows.at[p], table_sh.at[dbuf.at[j]], ssem[p], add=True)
        s_desc[(CHKB - 2) % NBUF].wait()
        s_desc[(CHKB - 1) % NBUF].wait()
        return carry

    lax.fori_loop(0, NCHK, main_chunk, None)

    deg_row0 = c * (E // 2 // BE) + s * (DPT // BE)

    def deg_chunk(k, carry):
        row0 = deg_row0 + k * CHKB
        pltpu.sync_copy(eidx.at[1, pl.ds(row0, CHKB)], dbuf)
        s_desc = [None] * NBUF
        for j in range(CHKB):
            p = j % NBUF
            if j >= NBUF:
                s_desc[p].wait()
            s_desc[p] = pltpu.async_copy(
                ones_v, deg_sh.at[dbuf.at[j]], ssem[p], add=True)
        for p in range(NBUF):
            s_desc[(CHKB - NBUF + p) % NBUF].wait()
        return carry

    lax.fori_loop(0, NDCHK, deg_chunk, None)
    plsc.subcore_barrier()

    @pl.when(s < 15)
    def _():
        st = s * DROWS_A
        pltpu.sync_copy(table_sh.at[pl.ds(st, DROWS_A)],
                        agg_out.at[pl.ds(coff + st, DROWS_A)])
        pltpu.sync_copy(deg_sh.at[pl.ds(st, DROWS_A)],
                        deg_out.at[pl.ds(coff + st, DROWS_A)])

    @pl.when(s == 15)
    def _():
        st = 15 * DROWS_A
        pltpu.sync_copy(table_sh.at[pl.ds(st, DROWS_B)],
                        agg_out.at[pl.ds(coff + st, DROWS_B)])
        pltpu.sync_copy(deg_sh.at[pl.ds(st, DROWS_B)],
                        deg_out.at[pl.ds(coff + st, DROWS_B)])


_conv_sc = pl.kernel(
    _conv_body,
    out_type=(
        jax.ShapeDtypeStruct((2 * N, 16), jnp.float32),
        jax.ShapeDtypeStruct((2 * N,), jnp.float32),
    ),
    mesh=plsc.VectorSubcoreMesh(core_axis_name="c", subcore_axis_name="s"),
    compiler_params=pltpu.CompilerParams(use_tc_tiling_on_sc=False),
    scratch_types=[
        pltpu.VMEM_SHARED((N_PAD, 16), jnp.float32),
        pltpu.VMEM_SHARED((N_PAD,), jnp.float32),
        pltpu.VMEM((CHKB, BE), jnp.int32),
        pltpu.VMEM((CHKB, BE), jnp.int32),
        pltpu.VMEM((NBUF, BE, 16), jnp.float32),
        pltpu.VMEM((BE,), jnp.float32),
        pltpu.SemaphoreType.DMA, pltpu.SemaphoreType.DMA,
        pltpu.SemaphoreType.DMA, pltpu.SemaphoreType.DMA,
        pltpu.SemaphoreType.DMA, pltpu.SemaphoreType.DMA,
        pltpu.SemaphoreType.DMA, pltpu.SemaphoreType.DMA,
    ],
)


_F_BLK = 2000


def _fuse_body(hs_ref, ap_ref, an_ref, dp_ref, dn_ref,
               wps_ref, wpn_ref, wns_ref, wnn_ref, wpsi_ref, out_ref):
    f32 = jnp.float32
    hn = jnp.concatenate([hs_ref[0], hs_ref[1]], axis=1)
    degp = jnp.maximum(dp_ref[0] + dp_ref[1], 1.0)
    degn = jnp.maximum(dn_ref[0] + dn_ref[1], 1.0)
    aggp = jnp.concatenate([ap_ref[0], ap_ref[1]], axis=1) / degp
    aggn = jnp.concatenate([an_ref[0], an_ref[1]], axis=1) / degn

    dot = functools.partial(lax.dot_general, precision=lax.Precision.HIGHEST,
                            preferred_element_type=f32)
    mm = lambda a, b: dot(a, b, (((1,), (0,)), ((), ())))
    mmt = lambda a, b: dot(a, b, (((1,), (1,)), ((), ())))

    wpsi = wpsi_ref[...]
    hp = mm(hn, wps_ref[...]) + mm(aggp, wpn_ref[...])
    hm = mm(hn, wns_ref[...]) + mm(aggn, wnn_ref[...])
    z = mmt(hp, wpsi[:, :D]) + mmt(hm, wpsi[:, D:])
    out_ref[...] = jnp.clip(jnp.tanh(z) - DAMPING * hn, -50.0, 50.0)


def _fuse(hn2, aggp, degp, aggn, degn, Wp_self, Wp_nbr, Wn_self, Wn_nbr, W_psi):
    grid = N // _F_BLK
    split3 = pl.BlockSpec((2, _F_BLK, 16), lambda i: (0, i, 0))
    split2 = pl.BlockSpec((2, _F_BLK, 1), lambda i: (0, i, 0))
    wspec = pl.BlockSpec((D, D), lambda i: (0, 0))
    return pl.pallas_call(
        _fuse_body,
        grid=(grid,),
        in_specs=[split3, split3, split3, split2, split2,
                  wspec, wspec, wspec, wspec,
                  pl.BlockSpec((D, 2 * D), lambda i: (0, 0))],
        out_specs=pl.BlockSpec((_F_BLK, D), lambda i: (i, 0)),
        out_shape=jax.ShapeDtypeStruct((N, D), jnp.float32),
    )(hn2.reshape(2, N, 16), aggp.reshape(2, N, 16), aggn.reshape(2, N, 16),
      degp.reshape(2, N, 1), degn.reshape(2, N, 1),
      Wp_self, Wp_nbr, Wn_self, Wn_nbr, W_psi)


def kernel(t, h, edge_index_pos, edge_index_neg, gamma, beta,
           Wp_self, Wp_nbr, Wn_self, Wn_nbr, W_psi):
    hn2 = _layernorm_split(h, gamma, beta)
    z16 = jnp.zeros((ZROWS, 16), jnp.float32)
    z1 = jnp.zeros((ZROWS,), jnp.float32)
    aggp, degp = _conv_sc(hn2, edge_index_pos.reshape(2, E // BE, BE), z16, z1)
    aggn, degn = _conv_sc(hn2, edge_index_neg.reshape(2, E // BE, BE), z16, z1)
    return _fuse(hn2, aggp, degp, aggn, degn,
                 Wp_self, Wp_nbr, Wn_self, Wn_nbr, W_psi)


# speedup vs baseline: 11.3861x; 1.2106x over previous
import functools

import jax
import jax.numpy as jnp
from jax import lax
from jax.experimental import pallas as pl
from jax.experimental.pallas import tpu as pltpu
from jax.experimental.pallas import tpu_sc as plsc

N = 100000
D = 32
E = 1600000
DAMPING = 0.1
EPS = 1e-5

BE = 128
CHKB = 25
CHK_E = BE * CHKB
NSUB = 16
NCHK = E // CHK_E
NDCHK = E // 2 // CHK_E
NBUF = 4
ZROWS = 6272
N_PAD = NSUB * ZROWS
DROWS_A = 6248
DROWS_B = N - 15 * DROWS_A


_LN_BLK = 2000


def _ln_body(h_ref, g_ref, b_ref, lo_ref, hi_ref):
    x = h_ref[...]
    mean = jnp.mean(x, axis=1, keepdims=True)
    xc = x - mean
    var = jnp.mean(xc * xc, axis=1, keepdims=True)
    y = xc * jax.lax.rsqrt(var + EPS) * g_ref[...] + b_ref[...]
    lo_ref[...] = y[:, :16]
    hi_ref[...] = y[:, 16:]


def _layernorm_split(h, gamma, beta):
    grid = N // _LN_BLK
    half = pl.BlockSpec((_LN_BLK, 16), lambda i: (i, 0))
    return pl.pallas_call(
        _ln_body,
        grid=(grid,),
        in_specs=[
            pl.BlockSpec((_LN_BLK, D), lambda i: (i, 0)),
            pl.BlockSpec((1, D), lambda i: (0, 0)),
            pl.BlockSpec((1, D), lambda i: (0, 0)),
        ],
        out_specs=[half, half],
        out_shape=[jax.ShapeDtypeStruct((N, 16), jnp.float32),
                   jax.ShapeDtypeStruct((N, 16), jnp.float32)],
    )(h, gamma.reshape(1, D), beta.reshape(1, D))


def _conv_body(hn_lo, hn_hi, eidx, z16, z1, agg_out, deg_out,
               table_sh, deg_sh, sbuf, dbuf, dbatch, rows, ones_v,
               g0, g1, g2, g3, s0, s1, s2, s3):
    gsem = [g0, g1, g2, g3]
    ssem = [s0, s1, s2, s3]
    c = lax.axis_index("c")
    s = lax.axis_index("s")

    zbase = s * ZROWS
    pltpu.sync_copy(z16, table_sh.at[pl.ds(zbase, ZROWS)])
    pltpu.sync_copy(z1, deg_sh.at[pl.ds(zbase, ZROWS)])
    for j in range(BE // 16):
        ones_v[pl.ds(j * 16, 16)] = jnp.full((16,), 1.0, jnp.float32)
    plsc.subcore_barrier()

    def make_chunk(table):
        def main_chunk(k, carry):
            eoff = k * CHK_E
            pltpu.sync_copy(eidx.at[0, pl.ds(eoff, CHK_E)], sbuf)
            pltpu.sync_copy(eidx.at[1, pl.ds(eoff, CHK_E)], dbuf)
            g_desc = [None] * NBUF
            s_desc = [None] * NBUF

            def gather_start(j):
                p = j % NBUF
                g_desc[p] = pltpu.async_copy(
                    table.at[sbuf.at[pl.ds(j * BE, BE)]], rows.at[p], gsem[p])

            gather_start(0)
            gather_start(1)
            for j in range(CHKB):
                p = j % NBUF
                pn = (j + 2) % NBUF
                if j >= 2:
                    s_desc[pn].wait()
                if j + 2 < CHKB:
                    gather_start(j + 2)
                for q in range(BE // 16):
                    sl = pl.ds(q * 16, 16)
                    dbatch[p, sl] = dbuf[pl.ds(j * BE + q * 16, 16)]
                g_desc[p].wait()
                s_desc[p] = pltpu.async_copy(
                    rows.at[p], table_sh.at[dbatch.at[p]], ssem[p], add=True)
            s_desc[(CHKB - 2) % NBUF].wait()
            s_desc[(CHKB - 1) % NBUF].wait()
            return carry
        return main_chunk

    nchunks = (NCHK - s + NSUB - 1) // NSUB

    @pl.when(c == 0)
    def _():
        lax.fori_loop(0, nchunks,
                      lambda i, cr: make_chunk(hn_lo)(s + i * NSUB, cr), None)

    @pl.when(c == 1)
    def _():
        lax.fori_loop(0, nchunks,
                      lambda i, cr: make_chunk(hn_hi)(s + i * NSUB, cr), None)

    deg_chunk0 = c * NDCHK

    def deg_chunk(i, carry):
        eoff = (deg_chunk0 + s + i * NSUB) * CHK_E
        pltpu.sync_copy(eidx.at[1, pl.ds(eoff, CHK_E)], dbuf)
        s_desc = [None] * NBUF
        for j in range(CHKB):
            p = j % NBUF
            if j >= NBUF:
                s_desc[p].wait()
            for q in range(BE // 16):
                sl = pl.ds(q * 16, 16)
                dbatch[p, sl] = dbuf[pl.ds(j * BE + q * 16, 16)]
            s_desc[p] = pltpu.async_copy(
                ones_v, deg_sh.at[dbatch.at[p]], ssem[p], add=True)
        for p in range(NBUF):
            s_desc[(CHKB - NBUF + p) % NBUF].wait()
        return carry

    ndeg = (NDCHK - s + NSUB - 1) // NSUB
    lax.fori_loop(0, ndeg, deg_chunk, None)
    plsc.subcore_barrier()

    coff = c * N

    @pl.when(s < 15)
    def _():
        st = s * DROWS_A
        pltpu.sync_copy(table_sh.at[pl.ds(st, DROWS_A)],
                        agg_out.at[pl.ds(coff + st, DROWS_A)])
        pltpu.sync_copy(deg_sh.at[pl.ds(st, DROWS_A)],
                        deg_out.at[pl.ds(coff + st, DROWS_A)])

    @pl.when(s == 15)
    def _():
        st = 15 * DROWS_A
        pltpu.sync_copy(table_sh.at[pl.ds(st, DROWS_B)],
                        agg_out.at[pl.ds(coff + st, DROWS_B)])
        pltpu.sync_copy(deg_sh.at[pl.ds(st, DROWS_B)],
                        deg_out.at[pl.ds(coff + st, DROWS_B)])


_conv_sc = pl.kernel(
    _conv_body,
    out_type=(
        jax.ShapeDtypeStruct((2 * N, 16), jnp.float32),
        jax.ShapeDtypeStruct((2 * N,), jnp.float32),
    ),
    mesh=plsc.VectorSubcoreMesh(core_axis_name="c", subcore_axis_name="s"),
    compiler_params=pltpu.CompilerParams(use_tc_tiling_on_sc=False),
    scratch_types=[
        pltpu.VMEM_SHARED((N_PAD, 16), jnp.float32),
        pltpu.VMEM_SHARED((N_PAD,), jnp.float32),
        pltpu.VMEM((CHK_E,), jnp.int32),
        pltpu.VMEM((CHK_E,), jnp.int32),
        pltpu.VMEM((NBUF, BE), jnp.int32),
        pltpu.VMEM((NBUF, BE, 16), jnp.float32),
        pltpu.VMEM((BE,), jnp.float32),
        pltpu.SemaphoreType.DMA, pltpu.SemaphoreType.DMA,
        pltpu.SemaphoreType.DMA, pltpu.SemaphoreType.DMA,
        pltpu.SemaphoreType.DMA, pltpu.SemaphoreType.DMA,
        pltpu.SemaphoreType.DMA, pltpu.SemaphoreType.DMA,
    ],
)


_F_BLK = 2000


def _fuse_body(hlo_ref, hhi_ref, aplo_ref, aphi_ref, anlo_ref, anhi_ref,
               dp0_ref, dp1_ref, dn0_ref, dn1_ref,
               wps_ref, wpn_ref, wns_ref, wnn_ref, wpsi_ref, out_ref):
    f32 = jnp.float32
    hn = jnp.concatenate([hlo_ref[...], hhi_ref[...]], axis=1)
    degp = jnp.maximum(dp0_ref[...] + dp1_ref[...], 1.0)
    degn = jnp.maximum(dn0_ref[...] + dn1_ref[...], 1.0)
    aggp = jnp.concatenate([aplo_ref[...], aphi_ref[...]], axis=1) / degp
    aggn = jnp.concatenate([anlo_ref[...], anhi_ref[...]], axis=1) / degn

    dot = functools.partial(lax.dot_general, precision=lax.Precision.HIGHEST,
                            preferred_element_type=f32)
    mm = lambda a, b: dot(a, b, (((1,), (0,)), ((), ())))
    mmt = lambda a, b: dot(a, b, (((1,), (1,)), ((), ())))

    wpsi = wpsi_ref[...]
    hp = mm(hn, wps_ref[...]) + mm(aggp, wpn_ref[...])
    hm = mm(hn, wns_ref[...]) + mm(aggn, wnn_ref[...])
    z = mmt(hp, wpsi[:, :D]) + mmt(hm, wpsi[:, D:])
    out_ref[...] = jnp.clip(jnp.tanh(z) - DAMPING * hn, -50.0, 50.0)


def _fuse(hn_lo, hn_hi, aggp, degp, aggn, degn,
          Wp_self, Wp_nbr, Wn_self, Wn_nbr, W_psi):
    grid = N // _F_BLK
    lo16 = pl.BlockSpec((_F_BLK, 16), lambda i: (i, 0))
    hi16 = pl.BlockSpec((_F_BLK, 16), lambda i: (grid + i, 0))
    lo1 = pl.BlockSpec((_F_BLK, 1), lambda i: (i, 0))
    hi1 = pl.BlockSpec((_F_BLK, 1), lambda i: (grid + i, 0))
    wspec = pl.BlockSpec((D, D), lambda i: (0, 0))
    degp2 = degp.reshape(2 * N, 1)
    degn2 = degn.reshape(2 * N, 1)
    return pl.pallas_call(
        _fuse_body,
        grid=(grid,),
        in_specs=[lo16, lo16, lo16, hi16, lo16, hi16,
                  lo1, hi1, lo1, hi1,
                  wspec, wspec, wspec, wspec,
                  pl.BlockSpec((D, 2 * D), lambda i: (0, 0))],
        out_specs=pl.BlockSpec((_F_BLK, D), lambda i: (i, 0)),
        out_shape=jax.ShapeDtypeStruct((N, D), jnp.float32),
    )(hn_lo, hn_hi, aggp, aggp, aggn, aggn,
      degp2, degp2, degn2, degn2,
      Wp_self, Wp_nbr, Wn_self, Wn_nbr, W_psi)


def kernel(t, h, edge_index_pos, edge_index_neg, gamma, beta,
           Wp_self, Wp_nbr, Wn_self, Wn_nbr, W_psi):
    hn_lo, hn_hi = _layernorm_split(h, gamma, beta)
    z16 = jnp.zeros((ZROWS, 16), jnp.float32)
    z1 = jnp.zeros((ZROWS,), jnp.float32)
    aggp, degp = _conv_sc(hn_lo, hn_hi, edge_index_pos, z16, z1)
    aggn, degn = _conv_sc(hn_lo, hn_hi, edge_index_neg, z16, z1)
    return _fuse(hn_lo, hn_hi, aggp, degp, aggn, degn,
                 Wp_self, Wp_nbr, Wn_self, Wn_nbr, W_psi)


# speedup vs baseline: 12.5821x; 1.1050x over previous
import functools

import jax
import jax.numpy as jnp
from jax import lax
from jax.experimental import pallas as pl
from jax.experimental.pallas import tpu as pltpu
from jax.experimental.pallas import tpu_sc as plsc

N = 100000
D = 32
E = 1600000
DAMPING = 0.1
EPS = 1e-5

BE = 128
CHKB = 25
CHK_E = BE * CHKB
NSUB = 16
NCHK = E // CHK_E
NDCHK = E // 2 // CHK_E
NBUF = 4
ZROWS = 6272
N_PAD = NSUB * ZROWS
DROWS_A = 6248
DROWS_B = N - 15 * DROWS_A


_LN_BLK = 4000


def _ln_body(h_ref, g_ref, b_ref, lo_ref, hi_ref):
    x = h_ref[...]
    mean = jnp.mean(x, axis=1, keepdims=True)
    xc = x - mean
    var = jnp.mean(xc * xc, axis=1, keepdims=True)
    y = xc * jax.lax.rsqrt(var + EPS) * g_ref[...] + b_ref[...]
    lo_ref[...] = y[:, :16]
    hi_ref[...] = y[:, 16:]


def _layernorm_split(h, gamma, beta):
    grid = N // _LN_BLK
    half = pl.BlockSpec((_LN_BLK, 16), lambda i: (i, 0))
    return pl.pallas_call(
        _ln_body,
        grid=(grid,),
        in_specs=[
            pl.BlockSpec((_LN_BLK, D), lambda i: (i, 0)),
            pl.BlockSpec((1, D), lambda i: (0, 0)),
            pl.BlockSpec((1, D), lambda i: (0, 0)),
        ],
        out_specs=[half, half],
        out_shape=[jax.ShapeDtypeStruct((N, 16), jnp.float32),
                   jax.ShapeDtypeStruct((N, 16), jnp.float32)],
    )(h, gamma.reshape(1, D), beta.reshape(1, D))


def _conv_body(hn_lo, hn_hi, eidx, z16, z1, agg_out, deg_out,
               table_sh, deg_sh, sbuf, dbuf, dbatch, rows, ones_v,
               g0, g1, g2, g3, s0, s1, s2, s3):
    gsem = [g0, g1, g2, g3]
    ssem = [s0, s1, s2, s3]
    c = lax.axis_index("c")
    s = lax.axis_index("s")

    zbase = s * ZROWS
    pltpu.sync_copy(z16, table_sh.at[pl.ds(zbase, ZROWS)])
    pltpu.sync_copy(z1, deg_sh.at[pl.ds(zbase, ZROWS)])
    for j in range(BE // 16):
        ones_v[pl.ds(j * 16, 16)] = jnp.full((16,), 1.0, jnp.float32)
    plsc.subcore_barrier()

    def make_chunk(table):
        def main_chunk(k, carry):
            eoff = k * CHK_E
            pltpu.sync_copy(eidx.at[0, pl.ds(eoff, CHK_E)], sbuf)
            pltpu.sync_copy(eidx.at[1, pl.ds(eoff, CHK_E)], dbuf)
            g_desc = [None] * NBUF
            s_desc = [None] * NBUF

            def gather_start(j):
                p = j % NBUF
                g_desc[p] = pltpu.async_copy(
                    table.at[sbuf.at[pl.ds(j * BE, BE)]], rows.at[p], gsem[p])

            gather_start(0)
            gather_start(1)
            for j in range(CHKB):
                p = j % NBUF
                pn = (j + 2) % NBUF
                if j >= 2:
                    s_desc[pn].wait()
                if j + 2 < CHKB:
                    gather_start(j + 2)
                for q in range(BE // 16):
                    sl = pl.ds(q * 16, 16)
                    dbatch[p, sl] = dbuf[pl.ds(j * BE + q * 16, 16)]
                g_desc[p].wait()
                s_desc[p] = pltpu.async_copy(
                    rows.at[p], table_sh.at[dbatch.at[p]], ssem[p], add=True)
            s_desc[(CHKB - 2) % NBUF].wait()
            s_desc[(CHKB - 1) % NBUF].wait()
            return carry
        return main_chunk

    nchunks = (NCHK - s + NSUB - 1) // NSUB

    @pl.when(c == 0)
    def _():
        lax.fori_loop(0, nchunks,
                      lambda i, cr: make_chunk(hn_lo)(s + i * NSUB, cr), None)

    @pl.when(c == 1)
    def _():
        lax.fori_loop(0, nchunks,
                      lambda i, cr: make_chunk(hn_hi)(s + i * NSUB, cr), None)

    deg_chunk0 = c * NDCHK

    def deg_chunk(i, carry):
        eoff = (deg_chunk0 + s + i * NSUB) * CHK_E
        pltpu.sync_copy(eidx.at[1, pl.ds(eoff, CHK_E)], dbuf)
        s_desc = [None] * NBUF
        for j in range(CHKB):
            p = j % NBUF
            if j >= NBUF:
                s_desc[p].wait()
            for q in range(BE // 16):
                sl = pl.ds(q * 16, 16)
                dbatch[p, sl] = dbuf[pl.ds(j * BE + q * 16, 16)]
            s_desc[p] = pltpu.async_copy(
                ones_v, deg_sh.at[dbatch.at[p]], ssem[p], add=True)
        for p in range(NBUF):
            s_desc[(CHKB - NBUF + p) % NBUF].wait()
        return carry

    ndeg = (NDCHK - s + NSUB - 1) // NSUB
    lax.fori_loop(0, ndeg, deg_chunk, None)
    plsc.subcore_barrier()

    coff = c * N

    @pl.when(s < 15)
    def _():
        st = s * DROWS_A
        pltpu.sync_copy(table_sh.at[pl.ds(st, DROWS_A)],
                        agg_out.at[pl.ds(coff + st, DROWS_A)])
        pltpu.sync_copy(deg_sh.at[pl.ds(st, DROWS_A)],
                        deg_out.at[pl.ds(coff + st, DROWS_A)])

    @pl.when(s == 15)
    def _():
        st = 15 * DROWS_A
        pltpu.sync_copy(table_sh.at[pl.ds(st, DROWS_B)],
                        agg_out.at[pl.ds(coff + st, DROWS_B)])
        pltpu.sync_copy(deg_sh.at[pl.ds(st, DROWS_B)],
                        deg_out.at[pl.ds(coff + st, DROWS_B)])


_conv_sc = pl.kernel(
    _conv_body,
    out_type=(
        jax.ShapeDtypeStruct((2 * N, 16), jnp.float32),
        jax.ShapeDtypeStruct((2 * N,), jnp.float32),
    ),
    mesh=plsc.VectorSubcoreMesh(core_axis_name="c", subcore_axis_name="s"),
    compiler_params=pltpu.CompilerParams(use_tc_tiling_on_sc=False),
    scratch_types=[
        pltpu.VMEM_SHARED((N_PAD, 16), jnp.float32),
        pltpu.VMEM_SHARED((N_PAD,), jnp.float32),
        pltpu.VMEM((CHK_E,), jnp.int32),
        pltpu.VMEM((CHK_E,), jnp.int32),
        pltpu.VMEM((NBUF, BE), jnp.int32),
        pltpu.VMEM((NBUF, BE, 16), jnp.float32),
        pltpu.VMEM((BE,), jnp.float32),
        pltpu.SemaphoreType.DMA, pltpu.SemaphoreType.DMA,
        pltpu.SemaphoreType.DMA, pltpu.SemaphoreType.DMA,
        pltpu.SemaphoreType.DMA, pltpu.SemaphoreType.DMA,
        pltpu.SemaphoreType.DMA, pltpu.SemaphoreType.DMA,
    ],
)


_F_BLK = 2000


def _fuse_body(hlo_ref, hhi_ref, aplo_ref, aphi_ref, anlo_ref, anhi_ref,
               dp0_ref, dp1_ref, dn0_ref, dn1_ref,
               wps_ref, wpn_ref, wns_ref, wnn_ref, wpsi_ref, out_ref):
    f32 = jnp.float32
    dot = functools.partial(lax.dot_general, precision=lax.Precision.HIGHEST,
                            preferred_element_type=f32)
    mm = lambda a, b: dot(a, b, (((1,), (0,)), ((), ())))
    mmt = lambda a, b: dot(a, b, (((1,), (1,)), ((), ())))

    wpsi = wpsi_ref[...]
    m1 = mmt(wps_ref[...], wpsi[:, :D]) + mmt(wns_ref[...], wpsi[:, D:])
    m2 = mmt(wpn_ref[...], wpsi[:, :D])
    m3 = mmt(wnn_ref[...], wpsi[:, D:])

    degp = jnp.maximum(dp0_ref[...] + dp1_ref[...], 1.0)
    degn = jnp.maximum(dn0_ref[...] + dn1_ref[...], 1.0)
    hlo = hlo_ref[...]
    hhi = hhi_ref[...]
    z = (mm(hlo, m1[:16]) + mm(hhi, m1[16:])
         + (mm(aplo_ref[...], m2[:16]) + mm(aphi_ref[...], m2[16:])) / degp
         + (mm(anlo_ref[...], m3[:16]) + mm(anhi_ref[...], m3[16:])) / degn)
    dmg = DAMPING * jnp.concatenate([hlo, hhi], axis=1)
    out_ref[...] = jnp.clip(jnp.tanh(z) - dmg, -50.0, 50.0)


def _fuse(hn_lo, hn_hi, aggp, degp, aggn, degn,
          Wp_self, Wp_nbr, Wn_self, Wn_nbr, W_psi):
    grid = N // _F_BLK
    lo16 = pl.BlockSpec((_F_BLK, 16), lambda i: (i, 0))
    hi16 = pl.BlockSpec((_F_BLK, 16), lambda i: (grid + i, 0))
    lo1 = pl.BlockSpec((_F_BLK, 1), lambda i: (i, 0))
    hi1 = pl.BlockSpec((_F_BLK, 1), lambda i: (grid + i, 0))
    wspec = pl.BlockSpec((D, D), lambda i: (0, 0))
    degp2 = degp.reshape(2 * N, 1)
    degn2 = degn.reshape(2 * N, 1)
    return pl.pallas_call(
        _fuse_body,
        grid=(grid,),
        in_specs=[lo16, lo16, lo16, hi16, lo16, hi16,
                  lo1, hi1, lo1, hi1,
                  wspec, wspec, wspec, wspec,
                  pl.BlockSpec((D, 2 * D), lambda i: (0, 0))],
        out_specs=pl.BlockSpec((_F_BLK, D), lambda i: (i, 0)),
        out_shape=jax.ShapeDtypeStruct((N, D), jnp.float32),
    )(hn_lo, hn_hi, aggp, aggp, aggn, aggn,
      degp2, degp2, degn2, degn2,
      Wp_self, Wp_nbr, Wn_self, Wn_nbr, W_psi)


def kernel(t, h, edge_index_pos, edge_index_neg, gamma, beta,
           Wp_self, Wp_nbr, Wn_self, Wn_nbr, W_psi):
    hn_lo, hn_hi = _layernorm_split(h, gamma, beta)
    z16 = jnp.zeros((ZROWS, 16), jnp.float32)
    z1 = jnp.zeros((ZROWS,), jnp.float32)
    aggp, degp = _conv_sc(hn_lo, hn_hi, edge_index_pos, z16, z1)
    aggn, degn = _conv_sc(hn_lo, hn_hi, edge_index_neg, z16, z1)
    return _fuse(hn_lo, hn_hi, aggp, degp, aggn, degn,
                 Wp_self, Wp_nbr, Wn_self, Wn_nbr, W_psi)
